# Initial kernel scaffold; baseline (speedup 1.0000x reference)
import jax
import jax.numpy as jnp
from jax.experimental import pallas as pl


def kernel(x, edge_index, Wq1, bq1, Wk1, bk1, Wv1, bv1, Ws1, bs1, Wq2, bq2, Wk2, bk2, Wv2, bv2, Ws2, bs2, Wl, bl):
    raise NotImplementedError("write your pallas kernel here")


# speedup vs baseline: 9.2353x; 9.2353x over previous
import functools
import math

import jax
import jax.numpy as jnp
from jax import lax
from jax.experimental import pallas as pl
from jax.experimental.pallas import tpu as pltpu
from jax.experimental.pallas import tpu_sc as plsc

N = 10000
E = 320000
D = 128
NC = 2
NS = 16
NW = NC * NS
EW = E // NW
B = 80
NCH = EW // B
RPT = 624
TAIL = N - NS * RPT
INV_SQRT_C = 1.0 / math.sqrt(128.0)

_f32 = jnp.float32
_i32 = jnp.int32


def _edge_body(q_hbm, k_hbm, v_hbm, src_hbm, dst_hbm, zeros_hbm,
               out_hbm, outd_hbm,
               src_idx, dst_idx, qrows, krows, vrows,
               denom, ushared, sem):
    c = lax.axis_index("c")
    s = lax.axis_index("s")
    wid = c * NS + s

    zero16 = jnp.zeros((16,), _f32)

    def zden(i, carry):
        denom[pl.ds(i * 16, 16)] = zero16
        return carry
    lax.fori_loop(0, N // 16, zden, 0, unroll=False)

    pltpu.sync_copy(zeros_hbm.at[pl.ds(s * RPT, RPT)],
                    ushared.at[pl.ds(s * RPT, RPT)])

    @pl.when(s == NS - 1)
    def _():
        pltpu.sync_copy(zeros_hbm.at[pl.ds(NS * RPT, TAIL)],
                        ushared.at[pl.ds(NS * RPT, TAIL)])

    plsc.subcore_barrier()

    def chunk_body(i, carry):
        base = wid * EW + i * B
        pltpu.sync_copy(src_hbm.at[pl.ds(base, B)], src_idx)
        pltpu.sync_copy(dst_hbm.at[pl.ds(base, B)], dst_idx)
        cq = pltpu.async_copy(q_hbm.at[dst_idx], qrows, sem)
        ck = pltpu.async_copy(k_hbm.at[src_idx], krows, sem)
        cv = pltpu.async_copy(v_hbm.at[src_idx], vrows, sem)
        cq.wait()
        ck.wait()
        cv.wait()

        lane0 = lax.iota(_i32, 16) == 0

        def edge_body(e, carry2):
            acc0 = qrows[e, pl.ds(0, 16)] * krows[e, pl.ds(0, 16)]
            acc1 = qrows[e, pl.ds(16, 16)] * krows[e, pl.ds(16, 16)]
            acc2 = qrows[e, pl.ds(32, 16)] * krows[e, pl.ds(32, 16)]
            acc3 = qrows[e, pl.ds(48, 16)] * krows[e, pl.ds(48, 16)]
            for r in range(4, 8):
                a = r & 3
                p = qrows[e, pl.ds(r * 16, 16)] * krows[e, pl.ds(r * 16, 16)]
                if a == 0:
                    acc0 = acc0 + p
                elif a == 1:
                    acc1 = acc1 + p
                elif a == 2:
                    acc2 = acc2 + p
                else:
                    acc3 = acc3 + p
            acc = ((acc0 + acc1) + (acc2 + acc3)) * INV_SQRT_C
            s = jnp.sum(acc)
            w = jnp.exp(jnp.zeros((16,), _f32) + s)
            for r in range(8):
                vrows[e, pl.ds(r * 16, 16)] = (
                    vrows[e, pl.ds(r * 16, 16)] * w)
            dstv = plsc.load_gather(dst_idx, [jnp.zeros((16,), _i32) + e])
            plsc.addupdate_scatter(denom, [dstv], w, mask=lane0)
            return carry2

        lax.fori_loop(0, B, edge_body, 0, unroll=False)
        pltpu.sync_copy(vrows, ushared.at[dst_idx], add=True)
        return carry

    lax.fori_loop(0, NCH, chunk_body, 0, unroll=False)
    plsc.subcore_barrier()
    pltpu.sync_copy(ushared.at[pl.ds(s * RPT, RPT)],
                    out_hbm.at[c, pl.ds(s * RPT, RPT)])

    @pl.when(s == NS - 1)
    def _():
        pltpu.sync_copy(ushared.at[pl.ds(NS * RPT, TAIL)],
                        out_hbm.at[c, pl.ds(NS * RPT, TAIL)])

    pltpu.sync_copy(denom, outd_hbm.at[pl.ds(wid * N, N)])


_edge_kernel = functools.partial(
    pl.kernel,
    out_type=(jax.ShapeDtypeStruct((NC, N, D), _f32),
              jax.ShapeDtypeStruct((NW * N,), _f32)),
    mesh=plsc.VectorSubcoreMesh(core_axis_name="c", subcore_axis_name="s"),
    compiler_params=pltpu.CompilerParams(needs_layout_passes=False),
    scratch_types=[
        pltpu.VMEM((B,), _i32),
        pltpu.VMEM((B,), _i32),
        pltpu.VMEM((B, D), _f32),
        pltpu.VMEM((B, D), _f32),
        pltpu.VMEM((B, D), _f32),
        pltpu.VMEM((N,), _f32),
        pltpu.VMEM_SHARED((N, D), _f32),
        pltpu.SemaphoreType.DMA,
    ],
)(_edge_body)


_BN = 1000
_GRID = N // _BN


def _dot(a, b):
    return jax.lax.dot_general(a, b, (((1,), (0,)), ((), ())),
                               preferred_element_type=_f32,
                               precision=jax.lax.Precision.HIGHEST)


def _proj_body(x_ref, w0, w1, w2, w3, b0, b1, b2, b3, o0, o1, o2, o3):
    xb = x_ref[...]
    o0[...] = _dot(xb, w0[...]) + b0[...]
    o1[...] = _dot(xb, w1[...]) + b1[...]
    o2[...] = _dot(xb, w2[...]) + b2[...]
    o3[...] = _dot(xb, w3[...]) + b3[...]


def _attn_h(u0, u1, dm, sk):
    us = u0[...] + u1[...]
    den = jnp.sum(dm[...], axis=1, keepdims=True)
    return jnp.maximum(us / (den + 1e-16) + sk[...], 0.0)


def _comb_body(u0, u1, dm, sk, w0, w1, w2, w3, b0, b1, b2, b3,
               o0, o1, o2, o3):
    h = _attn_h(u0, u1, dm, sk)
    o0[...] = _dot(h, w0[...]) + b0[...]
    o1[...] = _dot(h, w1[...]) + b1[...]
    o2[...] = _dot(h, w2[...]) + b2[...]
    o3[...] = _dot(h, w3[...]) + b3[...]


def _final_body(u0, u1, dm, sk, wl, bl, out):
    h = _attn_h(u0, u1, dm, sk)
    out[...] = _dot(h, wl[...]) + bl[...]


_row_spec = pl.BlockSpec((_BN, D), lambda i: (i, 0))
_d_spec = pl.BlockSpec((_BN, NW), lambda i: (i, 0))
_w_spec = pl.BlockSpec((D, D), lambda i: (0, 0))
_b_spec = pl.BlockSpec((1, D), lambda i: (0, 0))
_o4 = [jax.ShapeDtypeStruct((N, D), _f32)] * 4

_proj_call = pl.pallas_call(
    _proj_body, grid=(_GRID,),
    in_specs=[_row_spec] + [_w_spec] * 4 + [_b_spec] * 4,
    out_specs=[_row_spec] * 4, out_shape=_o4)

_comb_call = pl.pallas_call(
    _comb_body, grid=(_GRID,),
    in_specs=([_row_spec, _row_spec, _d_spec, _row_spec]
              + [_w_spec] * 4 + [_b_spec] * 4),
    out_specs=[_row_spec] * 4, out_shape=_o4)

_final_call = pl.pallas_call(
    _final_body, grid=(_GRID,),
    in_specs=[_row_spec, _row_spec, _d_spec, _row_spec, _w_spec, _b_spec],
    out_specs=_row_spec, out_shape=jax.ShapeDtypeStruct((N, D), _f32))


def kernel(x, edge_index, Wq1, bq1, Wk1, bk1, Wv1, bv1, Ws1, bs1,
           Wq2, bq2, Wk2, bk2, Wv2, bv2, Ws2, bs2, Wl, bl):
    src = edge_index[0]
    dst = edge_index[1]
    zeros = jnp.zeros((N, D), _f32)
    b2 = lambda b: b.reshape(1, D)

    q1, k1, v1, s1 = _proj_call(x, Wq1, Wk1, Wv1, Ws1,
                                b2(bq1), b2(bk1), b2(bv1), b2(bs1))
    u1, d1 = _edge_kernel(q1, k1, v1, src, dst, zeros)
    q2, k2, v2, s2 = _comb_call(u1[0], u1[1], d1.reshape(NW, N).T, s1,
                                Wq2, Wk2, Wv2, Ws2,
                                b2(bq2), b2(bk2), b2(bv2), b2(bs2))
    u2, d2 = _edge_kernel(q2, k2, v2, src, dst, zeros)
    return _final_call(u2[0], u2[1], d2.reshape(NW, N).T, s2, Wl, b2(bl))
